# speedup vs baseline: 1.7259x; 1.7259x over previous
import functools

import jax
import jax.numpy as jnp
from jax.experimental import pallas as pl

DIM = 2048
NUM_EXPERTS = 64
TOP_K = 8
N_GROUPS = 8
GROUP_SIZE = NUM_EXPERTS // N_GROUPS
TOPK_GROUP = 4
ROUTED_SCALING_FACTOR = 2.5

NEG = -1e30


def _router_body(T, x_ref, w_ref, b_ref, idx_ref, wt_ref):
    logits = jnp.dot(x_ref[:], w_ref[:], preferred_element_type=jnp.float32)
    s = jax.nn.sigmoid(logits)
    sc = s + b_ref[:]

    lane = jax.lax.broadcasted_iota(jnp.int32, (T, NUM_EXPERTS), 1)
    grp_of_lane = lane // GROUP_SIZE

    gcols = []
    for g in range(N_GROUPS):
        vals = jnp.where(grp_of_lane == g, sc, NEG)
        m1 = jnp.max(vals, axis=1, keepdims=True)
        i1 = jnp.min(jnp.where(vals == m1, lane, NUM_EXPERTS), axis=1, keepdims=True)
        m2 = jnp.max(jnp.where(lane == i1, NEG, vals), axis=1, keepdims=True)
        gcols.append(m1 + m2)
    gs = jnp.concatenate(gcols, axis=1)

    glane = jax.lax.broadcasted_iota(jnp.int32, (T, N_GROUPS), 1)
    allowed = jnp.zeros((T, NUM_EXPERTS), jnp.bool_)
    gtmp = gs
    for _ in range(TOPK_GROUP):
        gm = jnp.max(gtmp, axis=1, keepdims=True)
        gi = jnp.min(jnp.where(gtmp == gm, glane, N_GROUPS), axis=1, keepdims=True)
        allowed = allowed | (grp_of_lane == gi)
        gtmp = jnp.where(glane == gi, NEG, gtmp)

    tmp = jnp.where(allowed, sc, 0.0)

    icols, wcols = [], []
    for _ in range(TOP_K):
        m = jnp.max(tmp, axis=1, keepdims=True)
        ei = jnp.min(jnp.where(tmp == m, lane, NUM_EXPERTS), axis=1, keepdims=True)
        sel = lane == ei
        w = jnp.max(jnp.where(sel, s, NEG), axis=1, keepdims=True)
        icols.append(ei)
        wcols.append(w)
        tmp = jnp.where(sel, NEG, tmp)
    topk_idx = jnp.concatenate(icols, axis=1)
    topk_w = jnp.concatenate(wcols, axis=1)

    denom = jnp.sum(topk_w, axis=1, keepdims=True) + 1e-20
    topk_w = topk_w / denom * ROUTED_SCALING_FACTOR

    idx_ref[:] = topk_idx
    wt_ref[:] = topk_w


def kernel(x, W_gate, e_score_correction_bias):
    n = x.shape[0]
    T = 512
    grid = n // T
    wt = W_gate.T
    b2 = e_score_correction_bias.reshape(1, NUM_EXPERTS)
    return pl.pallas_call(
        functools.partial(_router_body, T),
        grid=(grid,),
        in_specs=[
            pl.BlockSpec((T, DIM), lambda i: (i, 0)),
            pl.BlockSpec((DIM, NUM_EXPERTS), lambda i: (0, 0)),
            pl.BlockSpec((1, NUM_EXPERTS), lambda i: (0, 0)),
        ],
        out_specs=[
            pl.BlockSpec((T, TOP_K), lambda i: (i, 0)),
            pl.BlockSpec((T, TOP_K), lambda i: (i, 0)),
        ],
        out_shape=[
            jax.ShapeDtypeStruct((n, TOP_K), jnp.int32),
            jax.ShapeDtypeStruct((n, TOP_K), jnp.float32),
        ],
    )(x, wt, b2)


# speedup vs baseline: 7.1399x; 4.1370x over previous
import functools

import jax
import jax.numpy as jnp
from jax.experimental import pallas as pl

DIM = 2048
NUM_EXPERTS = 64
TOP_K = 8
N_GROUPS = 8
GROUP_SIZE = NUM_EXPERTS // N_GROUPS
TOPK_GROUP = 4
ROUTED_SCALING_FACTOR = 2.5

NEG = -1e30


def _router_body(T, w_ref, x_ref, b_ref, idx_ref, wt_ref):
    logits = jax.lax.dot_general(
        w_ref[:], x_ref[:], (((1,), (1,)), ((), ())),
        preferred_element_type=jnp.float32)
    s = jax.nn.sigmoid(logits)
    sc = s + b_ref[:]

    gcols = []
    for g in range(N_GROUPS):
        vals = sc[g * GROUP_SIZE:(g + 1) * GROUP_SIZE, :]
        m1 = jnp.max(vals, axis=0, keepdims=True)
        eq = vals == m1
        cnt = jnp.sum(eq.astype(jnp.float32), axis=0, keepdims=True)
        m2 = jnp.max(jnp.where(eq, NEG, vals), axis=0, keepdims=True)
        gcols.append(m1 + jnp.where(cnt >= 2.0, m1, m2))
    gs = jnp.concatenate(gcols, axis=0)

    grow = jax.lax.broadcasted_iota(jnp.int32, (N_GROUPS, T), 0)
    sel_groups = jnp.zeros((N_GROUPS, T), jnp.bool_)
    gtmp = gs
    for _ in range(TOPK_GROUP):
        gm = jnp.max(gtmp, axis=0, keepdims=True)
        gi = jnp.min(jnp.where(gtmp == gm, grow, N_GROUPS), axis=0,
                     keepdims=True)
        hit = grow == gi
        sel_groups = sel_groups | hit
        gtmp = jnp.where(hit, NEG, gtmp)

    allowed = jnp.concatenate(
        [jnp.broadcast_to(sel_groups[g:g + 1, :], (GROUP_SIZE, T))
         for g in range(N_GROUPS)], axis=0)
    tmp = jnp.where(allowed, sc, 0.0)

    erow = jax.lax.broadcasted_iota(jnp.int32, (NUM_EXPERTS, T), 0)
    icols, wcols = [], []
    for _ in range(TOP_K):
        m = jnp.max(tmp, axis=0, keepdims=True)
        ei = jnp.min(jnp.where(tmp == m, erow, NUM_EXPERTS), axis=0,
                     keepdims=True)
        sel = erow == ei
        w = jnp.max(jnp.where(sel, s, NEG), axis=0, keepdims=True)
        icols.append(ei)
        wcols.append(w)
        tmp = jnp.where(sel, NEG, tmp)
    topk_idx = jnp.concatenate(icols, axis=0)
    topk_w = jnp.concatenate(wcols, axis=0)

    denom = jnp.sum(topk_w, axis=0, keepdims=True) + 1e-20
    topk_w = topk_w / denom * ROUTED_SCALING_FACTOR

    idx_ref[:] = topk_idx
    wt_ref[:] = topk_w


def kernel(x, W_gate, e_score_correction_bias):
    n = x.shape[0]
    T = 512
    grid = n // T
    b2 = e_score_correction_bias.reshape(NUM_EXPERTS, 1)
    idx_t, wt_t = pl.pallas_call(
        functools.partial(_router_body, T),
        grid=(grid,),
        in_specs=[
            pl.BlockSpec((NUM_EXPERTS, DIM), lambda i: (0, 0)),
            pl.BlockSpec((T, DIM), lambda i: (i, 0)),
            pl.BlockSpec((NUM_EXPERTS, 1), lambda i: (0, 0)),
        ],
        out_specs=[
            pl.BlockSpec((TOP_K, T), lambda i: (0, i)),
            pl.BlockSpec((TOP_K, T), lambda i: (0, i)),
        ],
        out_shape=[
            jax.ShapeDtypeStruct((TOP_K, n), jnp.int32),
            jax.ShapeDtypeStruct((TOP_K, n), jnp.float32),
        ],
    )(W_gate, x, b2)
    return idx_t.T, wt_t.T


# speedup vs baseline: 8.4130x; 1.1783x over previous
import functools

import jax
import jax.numpy as jnp
from jax.experimental import pallas as pl

DIM = 2048
NUM_EXPERTS = 64
TOP_K = 8
N_GROUPS = 8
GROUP_SIZE = NUM_EXPERTS // N_GROUPS
TOPK_GROUP = 4
ROUTED_SCALING_FACTOR = 2.5

NEG = -1e30


def _router_body(T, w_ref, x_ref, b_ref, idx_ref, wt_ref):
    logits = jax.lax.dot_general(
        w_ref[:], x_ref[:], (((1,), (1,)), ((), ())),
        preferred_element_type=jnp.float32)
    s = jax.nn.sigmoid(logits)
    sc = s + b_ref[:]

    gcols = []
    for g in range(N_GROUPS):
        vals = sc[g * GROUP_SIZE:(g + 1) * GROUP_SIZE, :]
        m1 = jnp.max(vals, axis=0, keepdims=True)
        eq = vals == m1
        cnt = jnp.sum(eq.astype(jnp.float32), axis=0, keepdims=True)
        m2 = jnp.max(jnp.where(eq, NEG, vals), axis=0, keepdims=True)
        gcols.append(m1 + jnp.where(cnt >= 2.0, m1, m2))
    gs = jnp.concatenate(gcols, axis=0)

    grow = jax.lax.broadcasted_iota(jnp.int32, (N_GROUPS, T), 0)
    sel_groups = jnp.zeros((N_GROUPS, T), jnp.bool_)
    gtmp = gs
    for _ in range(TOPK_GROUP):
        gm = jnp.max(gtmp, axis=0, keepdims=True)
        gi = jnp.min(jnp.where(gtmp == gm, grow, N_GROUPS), axis=0,
                     keepdims=True)
        hit = grow == gi
        sel_groups = sel_groups | hit
        gtmp = jnp.where(hit, NEG, gtmp)

    allowed = jnp.concatenate(
        [jnp.broadcast_to(sel_groups[g:g + 1, :], (GROUP_SIZE, T))
         for g in range(N_GROUPS)], axis=0)
    tmp = jnp.where(allowed, sc, 0.0)

    erow = jax.lax.broadcasted_iota(jnp.int32, (NUM_EXPERTS, T), 0)
    icols, wcols = [], []
    for _ in range(TOP_K):
        m = jnp.max(tmp, axis=0, keepdims=True)
        ei = jnp.min(jnp.where(tmp == m, erow, NUM_EXPERTS), axis=0,
                     keepdims=True)
        sel = erow == ei
        w = jnp.max(jnp.where(sel, s, NEG), axis=0, keepdims=True)
        icols.append(ei)
        wcols.append(w)
        tmp = jnp.where(sel, NEG, tmp)
    topk_idx = jnp.concatenate(icols, axis=0)
    topk_w = jnp.concatenate(wcols, axis=0)

    denom = jnp.sum(topk_w, axis=0, keepdims=True) + 1e-20
    topk_w = topk_w / denom * ROUTED_SCALING_FACTOR

    idx_ref[:] = topk_idx
    wt_ref[:] = topk_w


def kernel(x, W_gate, e_score_correction_bias):
    n = x.shape[0]
    T = 1024
    grid = n // T
    b2 = e_score_correction_bias.reshape(NUM_EXPERTS, 1)
    idx_t, wt_t = pl.pallas_call(
        functools.partial(_router_body, T),
        grid=(grid,),
        in_specs=[
            pl.BlockSpec((NUM_EXPERTS, DIM), lambda i: (0, 0)),
            pl.BlockSpec((T, DIM), lambda i: (i, 0)),
            pl.BlockSpec((NUM_EXPERTS, 1), lambda i: (0, 0)),
        ],
        out_specs=[
            pl.BlockSpec((TOP_K, T), lambda i: (0, i)),
            pl.BlockSpec((TOP_K, T), lambda i: (0, i)),
        ],
        out_shape=[
            jax.ShapeDtypeStruct((TOP_K, n), jnp.int32),
            jax.ShapeDtypeStruct((TOP_K, n), jnp.float32),
        ],
    )(W_gate, x, b2)
    return idx_t.T, wt_t.T


# speedup vs baseline: 9.1640x; 1.0893x over previous
import functools

import jax
import jax.numpy as jnp
from jax.experimental import pallas as pl

DIM = 2048
NUM_EXPERTS = 64
TOP_K = 8
N_GROUPS = 8
GROUP_SIZE = NUM_EXPERTS // N_GROUPS
TOPK_GROUP = 4
ROUTED_SCALING_FACTOR = 2.5

NEG = -1e30


def _router_body(T, w_ref, x_ref, b_ref, idx_ref, wt_ref):
    logits = jax.lax.dot_general(
        w_ref[:], x_ref[:], (((1,), (1,)), ((), ())),
        preferred_element_type=jnp.float32)
    s = jax.nn.sigmoid(logits)
    sc = s + b_ref[:]

    gcols = []
    for g in range(N_GROUPS):
        vals = sc[g * GROUP_SIZE:(g + 1) * GROUP_SIZE, :]
        m1 = jnp.max(vals, axis=0, keepdims=True)
        eq = vals == m1
        cnt = jnp.sum(eq.astype(jnp.float32), axis=0, keepdims=True)
        m2 = jnp.max(jnp.where(eq, NEG, vals), axis=0, keepdims=True)
        gcols.append(m1 + jnp.where(cnt >= 2.0, m1, m2))
    gs = jnp.concatenate(gcols, axis=0)

    grow = jax.lax.broadcasted_iota(jnp.int32, (N_GROUPS, T), 0)
    sel_groups = jnp.zeros((N_GROUPS, T), jnp.bool_)
    gtmp = gs
    for _ in range(TOPK_GROUP):
        gm = jnp.max(gtmp, axis=0, keepdims=True)
        gi = jnp.min(jnp.where(gtmp == gm, grow, N_GROUPS), axis=0,
                     keepdims=True)
        hit = grow == gi
        sel_groups = sel_groups | hit
        gtmp = jnp.where(hit, NEG, gtmp)

    allowed = jnp.concatenate(
        [jnp.broadcast_to(sel_groups[g:g + 1, :], (GROUP_SIZE, T))
         for g in range(N_GROUPS)], axis=0)
    tmp = jnp.where(allowed, sc, 0.0)

    erow = jax.lax.broadcasted_iota(jnp.int32, (NUM_EXPERTS, T), 0)
    icols, wcols = [], []
    for _ in range(TOP_K):
        m = jnp.max(tmp, axis=0, keepdims=True)
        ei = jnp.min(jnp.where(tmp == m, erow, NUM_EXPERTS), axis=0,
                     keepdims=True)
        sel = erow == ei
        w = jnp.max(jnp.where(sel, s, NEG), axis=0, keepdims=True)
        icols.append(ei)
        wcols.append(w)
        tmp = jnp.where(sel, NEG, tmp)
    topk_idx = jnp.concatenate(icols, axis=0)
    topk_w = jnp.concatenate(wcols, axis=0)

    denom = jnp.sum(topk_w, axis=0, keepdims=True) + 1e-20
    topk_w = topk_w / denom * ROUTED_SCALING_FACTOR

    idx_ref[:] = topk_idx
    wt_ref[:] = topk_w


def kernel(x, W_gate, e_score_correction_bias):
    n = x.shape[0]
    T = 2048
    grid = n // T
    b2 = e_score_correction_bias.reshape(NUM_EXPERTS, 1)
    idx_t, wt_t = pl.pallas_call(
        functools.partial(_router_body, T),
        grid=(grid,),
        in_specs=[
            pl.BlockSpec((NUM_EXPERTS, DIM), lambda i: (0, 0)),
            pl.BlockSpec((T, DIM), lambda i: (i, 0)),
            pl.BlockSpec((NUM_EXPERTS, 1), lambda i: (0, 0)),
        ],
        out_specs=[
            pl.BlockSpec((TOP_K, T), lambda i: (0, i)),
            pl.BlockSpec((TOP_K, T), lambda i: (0, i)),
        ],
        out_shape=[
            jax.ShapeDtypeStruct((TOP_K, n), jnp.int32),
            jax.ShapeDtypeStruct((TOP_K, n), jnp.float32),
        ],
    )(W_gate, x, b2)
    return idx_t.T, wt_t.T
